# baseline (speedup 1.0000x reference)
import jax
import jax.numpy as jnp
from jax.experimental import pallas as pl
from jax.experimental.pallas import tpu as pltpu


def _gather_kernel(x_hbm, o_ref, rows_v, sem):
    cp = pltpu.make_async_copy(x_hbm.at[pl.ds(1, 2), pl.ds(0, 128)], rows_v, sem)
    cp.start()
    cp.wait()
    o_ref[0] = rows_v[0, 2]
    o_ref[1] = rows_v[1, 1]


def kernel(x):
    return pl.pallas_call(
        _gather_kernel,
        grid=(1,),
        in_specs=[pl.BlockSpec(memory_space=pl.ANY)],
        out_specs=pl.BlockSpec(memory_space=pltpu.SMEM),
        out_shape=jax.ShapeDtypeStruct((2,), jnp.float32),
        scratch_shapes=[
            pltpu.VMEM((2, 128), jnp.float32),
            pltpu.SemaphoreType.DMA,
        ],
    )(x)


# speedup vs baseline: 1.1247x; 1.1247x over previous
import jax
import jax.numpy as jnp
from jax.experimental import pallas as pl
from jax.experimental.pallas import tpu as pltpu


def _gather_kernel(x_hbm, o_ref, rows_v, sem):
    cp = pltpu.make_async_copy(x_hbm.at[pl.ds(1, 2), pl.ds(0, 128)], rows_v, sem)
    cp.start()
    cp.wait()
    a = rows_v[0, 2]
    b = rows_v[1, 1]
    col = jax.lax.iota(jnp.int32, 2)
    o_ref[...] = jnp.where(col == 0, a, b)


def kernel(x):
    return pl.pallas_call(
        _gather_kernel,
        grid=(1,),
        in_specs=[pl.BlockSpec(memory_space=pl.ANY)],
        out_specs=pl.BlockSpec((2,), lambda i: (0,)),
        out_shape=jax.ShapeDtypeStruct((2,), jnp.float32),
        scratch_shapes=[
            pltpu.VMEM((2, 128), jnp.float32),
            pltpu.SemaphoreType.DMA,
        ],
    )(x)
